# baseline (speedup 1.0000x reference)
import jax
import jax.numpy as jnp
from jax import lax
from jax.experimental import pallas as pl
from jax.experimental.pallas import tpu as pltpu
from jax.experimental.pallas import tpu_sc as plsc

_NFEAT = 8
_NFIELD = 4
_FIELD_OF = [i % _NFIELD for i in range(_NFEAT)]
_DIM = 16
_BATCH = 4096
_VOCAB = 100000
_RPD = _VOCAB // _DIM

_NC, _NS = 2, 16
_NW = _NC * _NS
_BPW = _BATCH // _NW
_NCHUNK = _BPW // _DIM
_NTAB = _NFEAT * _NFIELD
_HDIM = _DIM // 2
_SPLIT_F = 6

_mesh = plsc.VectorSubcoreMesh(
    core_axis_name="c", subcore_axis_name="s",
    num_cores=_NC, num_subcores=_NS)

_CP = pltpu.CompilerParams(needs_layout_passes=False, use_tc_tiling_on_sc=False)


def _common_scratch(nfeat):
    return [
        *[pltpu.VMEM((_BPW,), jnp.int32) for _ in range(nfeat)],
        *[pltpu.VMEM((_BPW,), jnp.int32) for _ in range(nfeat)],
        *[pltpu.VMEM((_HDIM * _BPW,), jnp.int32) for _ in range(2)],
        *[pltpu.VMEM((_HDIM * _BPW, _DIM), jnp.float32) for _ in range(2)],
        pltpu.VMEM((_NFIELD * _NFIELD, _DIM, _BPW), jnp.float32),
        pltpu.VMEM((_NFIELD, _BPW), jnp.float32),
        pltpu.SemaphoreType.DMA,
    ]


def _stage_tables(feats, idx_v, linidx_v, ridx_v, jbuf_v, h_v, norm_v, sem,
                  emb_hbm, tables, lanes, any_first):
    def build_ridx(f, buf):
        def body(cb, _):
            sl = pl.ds(cb * _DIM, _DIM)
            rb = linidx_v[f][sl]
            for j in range(_HDIM):
                buf[pl.ds(j * _BPW + cb * _DIM, _DIM)] = rb + (j * _RPD)
            return 0
        lax.fori_loop(0, _NCHUNK, body, 0)

    nt = len(tables)

    def fire(k):
        fk = tables[k][1]
        return pltpu.async_copy(
            emb_hbm[tables[k][0]].at[ridx_v[(fk - tables[0][1]) % 2]],
            jbuf_v[k % 2], sem)

    build_ridx(tables[0][1], ridx_v[0])
    copies = [fire(0)]
    for k in range(nt):
        _, f, c, first, jh = tables[k]
        a = _FIELD_OF[f]
        if k + 1 < nt:
            fn = tables[k + 1][1]
            if fn != f:
                build_ridx(fn, ridx_v[(fn - tables[0][1]) % 2])
            copies.append(fire(k + 1))
        copies[k].wait()
        jbuf = jbuf_v[k % 2]

        def acc_body(cb, _, f=f, c=c, a=a, jbuf=jbuf, first=first, jh=jh):
            sl = pl.ds(cb * _DIM, _DIM)
            rows = cb * _DIM + lanes
            mod = idx_v[f][sl] & 15
            nacc = None
            for j in range(_HDIM):
                v = plsc.load_gather(jbuf, [j * _BPW + rows, mod])
                jd = jh * _HDIM + j
                if first:
                    h_v[a * _NFIELD + c, jd, sl] = v
                else:
                    h_v[a * _NFIELD + c, jd, sl] = h_v[a * _NFIELD + c, jd, sl] + v
                if c == a:
                    nacc = v * v if nacc is None else nacc + v * v
            if c == a:
                if first and jh == 0:
                    norm_v[a, sl] = nacc
                else:
                    norm_v[a, sl] = norm_v[a, sl] + nacc
            return 0

        lax.fori_loop(0, _NCHUNK, acc_body, 0)


def _body1(*refs):
    nf = _SPLIT_F
    feat_hbm = refs[0:nf]
    emb_hbm = refs[nf:nf + 2 * nf * _NFIELD]
    hout_hbm = refs[nf + 2 * nf * _NFIELD]
    nout_hbm = refs[nf + 2 * nf * _NFIELD + 1]
    off = nf + 2 * nf * _NFIELD + 2
    idx_v = refs[off:off + nf]
    linidx_v = refs[off + nf:off + 2 * nf]
    ridx_v = refs[off + 2 * nf:off + 2 * nf + 2]
    jbuf_v = refs[off + 2 * nf + 2:off + 2 * nf + 4]
    h_v, norm_v, sem = refs[off + 2 * nf + 4:off + 2 * nf + 7]

    wid = lax.axis_index("s") * _NC + lax.axis_index("c")
    base = wid * _BPW
    lanes = lax.iota(jnp.int32, _DIM)

    for f in range(nf):
        pltpu.sync_copy(feat_hbm[f].at[pl.ds(base, _BPW)], idx_v[f])

    def linidx_body(cb, _):
        sl = pl.ds(cb * _DIM, _DIM)
        for f in range(nf):
            linidx_v[f][sl] = idx_v[f][sl] >> 4
        return 0

    lax.fori_loop(0, _NCHUNK, linidx_body, 0)

    tables = [(2 * (f * _NFIELD + c) + h, f, c, f < _NFIELD, h)
              for f in range(nf) for c in range(_NFIELD) for h in range(2)]
    _stage_tables(None, idx_v, linidx_v, ridx_v, jbuf_v, h_v, norm_v, sem,
                  emb_hbm, tables, lanes, True)

    pltpu.sync_copy(h_v, hout_hbm.at[wid])
    pltpu.sync_copy(norm_v, nout_hbm.at[wid])


def _body2(*refs):
    nf = _NFEAT - _SPLIT_F
    bias_hbm = refs[0]
    allfeat_hbm = refs[1:1 + _NFEAT]
    lin_hbm = refs[1 + _NFEAT:1 + 2 * _NFEAT]
    emb_hbm = refs[1 + 2 * _NFEAT:1 + 2 * _NFEAT + 2 * nf * _NFIELD]
    hin_hbm = refs[1 + 2 * _NFEAT + 2 * nf * _NFIELD]
    nin_hbm = refs[2 + 2 * _NFEAT + 2 * nf * _NFIELD]
    out_hbm = refs[3 + 2 * _NFEAT + 2 * nf * _NFIELD]
    off = 4 + 2 * _NFEAT + 2 * nf * _NFIELD
    idx_v = refs[off:off + _NFEAT]
    linidx_v = refs[off + _NFEAT:off + 2 * _NFEAT]
    ridx_v = refs[off + 2 * _NFEAT:off + 2 * _NFEAT + 2]
    jbuf_v = refs[off + 2 * _NFEAT + 2:off + 2 * _NFEAT + 4]
    h_v, norm_v, sem = refs[off + 2 * _NFEAT + 4:off + 2 * _NFEAT + 7]
    lin_v, out_v, bias_v, sem2 = refs[off + 2 * _NFEAT + 7:off + 2 * _NFEAT + 11]

    wid = lax.axis_index("s") * _NC + lax.axis_index("c")
    base = wid * _BPW
    lanes = lax.iota(jnp.int32, _DIM)

    pltpu.sync_copy(bias_hbm, bias_v)
    for f in range(_NFEAT):
        pltpu.sync_copy(allfeat_hbm[f].at[pl.ds(base, _BPW)], idx_v[f])

    def linidx_body(cb, _):
        sl = pl.ds(cb * _DIM, _DIM)
        for f in range(_NFEAT):
            linidx_v[f][sl] = idx_v[f][sl] >> 4
        return 0

    lax.fori_loop(0, _NCHUNK, linidx_body, 0)

    lin_copies = [
        pltpu.async_copy(lin_hbm[f].at[linidx_v[f]], lin_v.at[f], sem2)
        for f in range(_NFEAT)
    ]
    pltpu.sync_copy(hin_hbm.at[wid], h_v)
    pltpu.sync_copy(nin_hbm.at[wid], norm_v)

    tables = [(2 * ((f - _SPLIT_F) * _NFIELD + c) + h, f, c, False, h)
              for f in range(_SPLIT_F, _NFEAT) for c in range(_NFIELD)
              for h in range(2)]
    _stage_tables(None, idx_v, linidx_v, ridx_v, jbuf_v, h_v, norm_v, sem,
                  emb_hbm, tables, lanes, False)

    for cp in lin_copies:
        cp.wait()

    bias_bc = bias_v[:]

    def red_body(cb, _):
        sl = pl.ds(cb * _DIM, _DIM)
        rows = cb * _DIM + lanes
        tot = bias_bc
        for a in range(_NFIELD):
            for b2 in range(a + 1, _NFIELD):
                for j in range(_DIM):
                    tot = tot + (h_v[a * _NFIELD + b2, j, sl]
                                 * h_v[b2 * _NFIELD + a, j, sl])
        sq = None
        for a in range(_NFIELD):
            for j in range(_DIM):
                h = h_v[a * _NFIELD + a, j, sl]
                sq = h * h if j == 0 else sq + h * h
            tot = tot + 0.5 * (sq - norm_v[a, sl])
        for f in range(_NFEAT):
            mod = idx_v[f][sl] & 15
            tot = tot + plsc.load_gather(
                lin_v, [jnp.full((_DIM,), f, jnp.int32), rows, mod])
        out_v[sl] = tot
        return 0

    lax.fori_loop(0, _NCHUNK, red_body, 0)
    pltpu.sync_copy(out_v, out_hbm.at[pl.ds(base, _BPW)])


_call1 = pl.kernel(
    _body1,
    mesh=_mesh,
    out_type=(
        jax.ShapeDtypeStruct((_NW, _NFIELD * _NFIELD, _DIM, _BPW), jnp.float32),
        jax.ShapeDtypeStruct((_NW, _NFIELD, _BPW), jnp.float32),
    ),
    scratch_types=_common_scratch(_SPLIT_F),
    compiler_params=_CP,
)

_call2 = pl.kernel(
    _body2,
    mesh=_mesh,
    out_type=jax.ShapeDtypeStruct((_BATCH,), jnp.float32),
    scratch_types=_common_scratch(_NFEAT) + [
        pltpu.VMEM((_NFEAT, _BPW, _DIM), jnp.float32),
        pltpu.VMEM((_BPW,), jnp.float32),
        pltpu.VMEM((_DIM,), jnp.float32),
        pltpu.SemaphoreType.DMA,
    ],
    compiler_params=_CP,
)


def kernel(bias, feat_0, feat_1, feat_2, feat_3, feat_4, feat_5, feat_6, feat_7, lin_feat_0, lin_feat_1, lin_feat_2, lin_feat_3, lin_feat_4, lin_feat_5, lin_feat_6, lin_feat_7, emb_feat_0_field_0, emb_feat_0_field_1, emb_feat_0_field_2, emb_feat_0_field_3, emb_feat_1_field_0, emb_feat_1_field_1, emb_feat_1_field_2, emb_feat_1_field_3, emb_feat_2_field_0, emb_feat_2_field_1, emb_feat_2_field_2, emb_feat_2_field_3, emb_feat_3_field_0, emb_feat_3_field_1, emb_feat_3_field_2, emb_feat_3_field_3, emb_feat_4_field_0, emb_feat_4_field_1, emb_feat_4_field_2, emb_feat_4_field_3, emb_feat_5_field_0, emb_feat_5_field_1, emb_feat_5_field_2, emb_feat_5_field_3, emb_feat_6_field_0, emb_feat_6_field_1, emb_feat_6_field_2, emb_feat_6_field_3, emb_feat_7_field_0, emb_feat_7_field_1, emb_feat_7_field_2, emb_feat_7_field_3):
    args = locals()
    feats = [args[f"feat_{i}"].astype(jnp.int32) for i in range(_NFEAT)]
    lins = [args[f"lin_feat_{i}"].reshape(-1, _DIM) for i in range(_NFEAT)]
    embs = []
    for i in range(_NFEAT):
        for c in range(_NFIELD):
            et = args[f"emb_feat_{i}_field_{c}"].T
            embs.append(et[:_HDIM].reshape(-1, _DIM))
            embs.append(et[_HDIM:].reshape(-1, _DIM))
    bias16 = jnp.broadcast_to(bias.astype(jnp.float32), (_DIM,))
    h_part, n_part = _call1(*feats[:_SPLIT_F], *embs[:2 * _SPLIT_F * _NFIELD])
    return _call2(bias16, *feats, *lins,
                  *embs[2 * _SPLIT_F * _NFIELD:], h_part, n_part)


# speedup vs baseline: 1.3938x; 1.3938x over previous
import jax
import jax.numpy as jnp
from jax import lax
from jax.experimental import pallas as pl
from jax.experimental.pallas import tpu as pltpu
from jax.experimental.pallas import tpu_sc as plsc

_NFEAT = 8
_NFIELD = 4
_FIELD_OF = [i % _NFIELD for i in range(_NFEAT)]
_DIM = 16
_BATCH = 4096
_VOCAB = 100000
_RPD = _VOCAB // _DIM

_NC, _NS = 2, 16
_NW = _NC * _NS
_BPW = _BATCH // _NW
_NCHUNK = _BPW // _DIM
_NTAB = _NFEAT * _NFIELD
_SPLIT_F = 6

_mesh = plsc.VectorSubcoreMesh(
    core_axis_name="c", subcore_axis_name="s",
    num_cores=_NC, num_subcores=_NS)

_CP = pltpu.CompilerParams(needs_layout_passes=False, use_tc_tiling_on_sc=False)


def _common_scratch(nfeat):
    return [
        *[pltpu.VMEM((_BPW,), jnp.int32) for _ in range(nfeat)],
        *[pltpu.VMEM((_BPW,), jnp.int32) for _ in range(nfeat)],
        *[pltpu.VMEM((_DIM * _BPW,), jnp.int32) for _ in range(2)],
        *[pltpu.VMEM((_DIM * _BPW, _DIM), jnp.float32) for _ in range(2)],
        pltpu.VMEM((_NFIELD * _NFIELD, _DIM, _BPW), jnp.float32),
        pltpu.VMEM((_NFIELD, _BPW), jnp.float32),
        pltpu.SemaphoreType.DMA,
    ]


def _stage_tables(feats, idx_v, linidx_v, ridx_v, jbuf_v, h_v, norm_v, sem,
                  emb_hbm, tables, lanes, any_first):
    def build_ridx(f, buf):
        def body(cb, _):
            sl = pl.ds(cb * _DIM, _DIM)
            rb = linidx_v[f][sl]
            for j in range(_DIM):
                buf[pl.ds(j * _BPW + cb * _DIM, _DIM)] = rb + (j * _RPD)
            return 0
        lax.fori_loop(0, _NCHUNK, body, 0)

    nt = len(tables)

    def fire(k):
        fk = tables[k][1]
        return pltpu.async_copy(
            emb_hbm[tables[k][0]].at[ridx_v[(fk - tables[0][1]) % 2]],
            jbuf_v[k % 2], sem)

    build_ridx(tables[0][1], ridx_v[0])
    copies = [fire(0)]
    for k in range(nt):
        _, f, c, first = tables[k]
        a = _FIELD_OF[f]
        if k + 1 < nt:
            fn = tables[k + 1][1]
            if fn != f:
                build_ridx(fn, ridx_v[(fn - tables[0][1]) % 2])
            copies.append(fire(k + 1))
        copies[k].wait()
        jbuf = jbuf_v[k % 2]

        def acc_body(cb, _, f=f, c=c, a=a, jbuf=jbuf, first=first):
            sl = pl.ds(cb * _DIM, _DIM)
            rows = cb * _DIM + lanes
            mod = idx_v[f][sl] & 15
            nacc = None
            for j in range(_DIM):
                v = plsc.load_gather(jbuf, [j * _BPW + rows, mod])
                if first:
                    h_v[a * _NFIELD + c, j, sl] = v
                else:
                    h_v[a * _NFIELD + c, j, sl] = h_v[a * _NFIELD + c, j, sl] + v
                if c == a:
                    nacc = v * v if nacc is None else nacc + v * v
            if c == a:
                if first:
                    norm_v[a, sl] = nacc
                else:
                    norm_v[a, sl] = norm_v[a, sl] + nacc
            return 0

        lax.fori_loop(0, _NCHUNK, acc_body, 0)


def _body1(*refs):
    nf = _SPLIT_F
    feat_hbm = refs[0:nf]
    emb_hbm = refs[nf:nf + nf * _NFIELD]
    hout_hbm = refs[nf + nf * _NFIELD]
    nout_hbm = refs[nf + nf * _NFIELD + 1]
    off = nf + nf * _NFIELD + 2
    idx_v = refs[off:off + nf]
    linidx_v = refs[off + nf:off + 2 * nf]
    ridx_v = refs[off + 2 * nf:off + 2 * nf + 2]
    jbuf_v = refs[off + 2 * nf + 2:off + 2 * nf + 4]
    h_v, norm_v, sem = refs[off + 2 * nf + 4:off + 2 * nf + 7]

    wid = lax.axis_index("s") * _NC + lax.axis_index("c")
    base = wid * _BPW
    lanes = lax.iota(jnp.int32, _DIM)

    for f in range(nf):
        pltpu.sync_copy(feat_hbm[f].at[pl.ds(base, _BPW)], idx_v[f])

    def linidx_body(cb, _):
        sl = pl.ds(cb * _DIM, _DIM)
        for f in range(nf):
            linidx_v[f][sl] = idx_v[f][sl] >> 4
        return 0

    lax.fori_loop(0, _NCHUNK, linidx_body, 0)

    tables = [(f * _NFIELD + c, f, c, f < _NFIELD)
              for f in range(nf) for c in range(_NFIELD)]
    _stage_tables(None, idx_v, linidx_v, ridx_v, jbuf_v, h_v, norm_v, sem,
                  emb_hbm, tables, lanes, True)

    pltpu.sync_copy(h_v, hout_hbm.at[wid])
    pltpu.sync_copy(norm_v, nout_hbm.at[wid])


def _body2(*refs):
    nf = _NFEAT - _SPLIT_F
    bias_hbm = refs[0]
    allfeat_hbm = refs[1:1 + _NFEAT]
    lin_hbm = refs[1 + _NFEAT:1 + 2 * _NFEAT]
    emb_hbm = refs[1 + 2 * _NFEAT:1 + 2 * _NFEAT + nf * _NFIELD]
    hin_hbm = refs[1 + 2 * _NFEAT + nf * _NFIELD]
    nin_hbm = refs[2 + 2 * _NFEAT + nf * _NFIELD]
    out_hbm = refs[3 + 2 * _NFEAT + nf * _NFIELD]
    off = 4 + 2 * _NFEAT + nf * _NFIELD
    idx_v = refs[off:off + _NFEAT]
    linidx_v = refs[off + _NFEAT:off + 2 * _NFEAT]
    ridx_v = refs[off + 2 * _NFEAT:off + 2 * _NFEAT + 2]
    jbuf_v = refs[off + 2 * _NFEAT + 2:off + 2 * _NFEAT + 4]
    h_v, norm_v, sem = refs[off + 2 * _NFEAT + 4:off + 2 * _NFEAT + 7]
    lin_v, out_v, bias_v, sem2 = refs[off + 2 * _NFEAT + 7:off + 2 * _NFEAT + 11]

    wid = lax.axis_index("s") * _NC + lax.axis_index("c")
    base = wid * _BPW
    lanes = lax.iota(jnp.int32, _DIM)

    pltpu.sync_copy(bias_hbm, bias_v)
    for f in range(_NFEAT):
        pltpu.sync_copy(allfeat_hbm[f].at[pl.ds(base, _BPW)], idx_v[f])

    def linidx_body(cb, _):
        sl = pl.ds(cb * _DIM, _DIM)
        for f in range(_NFEAT):
            linidx_v[f][sl] = idx_v[f][sl] >> 4
        return 0

    lax.fori_loop(0, _NCHUNK, linidx_body, 0)

    lin_copies = [
        pltpu.async_copy(lin_hbm[f].at[linidx_v[f]], lin_v.at[f], sem2)
        for f in range(_NFEAT)
    ]
    pltpu.sync_copy(hin_hbm.at[wid], h_v)
    pltpu.sync_copy(nin_hbm.at[wid], norm_v)

    tables = [((f - _SPLIT_F) * _NFIELD + c, f, c, False)
              for f in range(_SPLIT_F, _NFEAT) for c in range(_NFIELD)]
    _stage_tables(None, idx_v, linidx_v, ridx_v, jbuf_v, h_v, norm_v, sem,
                  emb_hbm, tables, lanes, False)

    for cp in lin_copies:
        cp.wait()

    bias_bc = bias_v[:]

    def red_body(cb, _):
        sl = pl.ds(cb * _DIM, _DIM)
        rows = cb * _DIM + lanes
        tot = bias_bc
        for a in range(_NFIELD):
            for b2 in range(a + 1, _NFIELD):
                for j in range(_DIM):
                    tot = tot + (h_v[a * _NFIELD + b2, j, sl]
                                 * h_v[b2 * _NFIELD + a, j, sl])
        sq = None
        for a in range(_NFIELD):
            for j in range(_DIM):
                h = h_v[a * _NFIELD + a, j, sl]
                sq = h * h if j == 0 else sq + h * h
            tot = tot + 0.5 * (sq - norm_v[a, sl])
        for f in range(_NFEAT):
            mod = idx_v[f][sl] & 15
            tot = tot + plsc.load_gather(
                lin_v, [jnp.full((_DIM,), f, jnp.int32), rows, mod])
        out_v[sl] = tot
        return 0

    lax.fori_loop(0, _NCHUNK, red_body, 0)
    pltpu.sync_copy(out_v, out_hbm.at[pl.ds(base, _BPW)])


_call1 = pl.kernel(
    _body1,
    mesh=_mesh,
    out_type=(
        jax.ShapeDtypeStruct((_NW, _NFIELD * _NFIELD, _DIM, _BPW), jnp.float32),
        jax.ShapeDtypeStruct((_NW, _NFIELD, _BPW), jnp.float32),
    ),
    scratch_types=_common_scratch(_SPLIT_F),
    compiler_params=_CP,
)

_call2 = pl.kernel(
    _body2,
    mesh=_mesh,
    out_type=jax.ShapeDtypeStruct((_BATCH,), jnp.float32),
    scratch_types=_common_scratch(_NFEAT) + [
        pltpu.VMEM((_NFEAT, _BPW, _DIM), jnp.float32),
        pltpu.VMEM((_BPW,), jnp.float32),
        pltpu.VMEM((_DIM,), jnp.float32),
        pltpu.SemaphoreType.DMA,
    ],
    compiler_params=_CP,
)


def kernel(bias, feat_0, feat_1, feat_2, feat_3, feat_4, feat_5, feat_6, feat_7, lin_feat_0, lin_feat_1, lin_feat_2, lin_feat_3, lin_feat_4, lin_feat_5, lin_feat_6, lin_feat_7, emb_feat_0_field_0, emb_feat_0_field_1, emb_feat_0_field_2, emb_feat_0_field_3, emb_feat_1_field_0, emb_feat_1_field_1, emb_feat_1_field_2, emb_feat_1_field_3, emb_feat_2_field_0, emb_feat_2_field_1, emb_feat_2_field_2, emb_feat_2_field_3, emb_feat_3_field_0, emb_feat_3_field_1, emb_feat_3_field_2, emb_feat_3_field_3, emb_feat_4_field_0, emb_feat_4_field_1, emb_feat_4_field_2, emb_feat_4_field_3, emb_feat_5_field_0, emb_feat_5_field_1, emb_feat_5_field_2, emb_feat_5_field_3, emb_feat_6_field_0, emb_feat_6_field_1, emb_feat_6_field_2, emb_feat_6_field_3, emb_feat_7_field_0, emb_feat_7_field_1, emb_feat_7_field_2, emb_feat_7_field_3):
    args = locals()
    feats = [args[f"feat_{i}"].astype(jnp.int32) for i in range(_NFEAT)]
    lins = [args[f"lin_feat_{i}"].reshape(-1, _DIM) for i in range(_NFEAT)]
    embs = [args[f"emb_feat_{i}_field_{c}"].T.reshape(_VOCAB, _DIM)
            for i in range(_NFEAT) for c in range(_NFIELD)]
    bias16 = jnp.broadcast_to(bias.astype(jnp.float32), (_DIM,))
    h_part, n_part = _call1(*feats[:_SPLIT_F], *embs[:_SPLIT_F * _NFIELD])
    return _call2(bias16, *feats, *lins,
                  *embs[_SPLIT_F * _NFIELD:], h_part, n_part)
